# baseline (speedup 1.0000x reference)
import functools

import jax
import jax.numpy as jnp
from jax.experimental import pallas as pl
from jax.experimental.pallas import tpu as pltpu
from jax.experimental.pallas import tpu_sc as plsc

_LANES = 16


def kernel(token_ids, embedding_table):
    B, L = token_ids.shape
    D = embedding_table.shape[1]
    LB, BB = L // 8, B // 128

    idx4 = token_ids.reshape(BB, 128, LB, 8).transpose(2, 0, 3, 1)

    mesh = plsc.VectorSubcoreMesh(core_axis_name="c", subcore_axis_name="s")

    @functools.partial(
        pl.kernel,
        mesh=mesh,
        out_type=jax.ShapeDtypeStruct((L, D // 8, BB, 8, 128), jnp.float32),
        scratch_types=[pltpu.VMEM((128, D), jnp.float32)],
        compiler_params=pltpu.CompilerParams(use_tc_tiling_on_sc=False,
                                             needs_layout_passes=False),
    )
    def gather_kernel(tab_hbm, idx_hbm, out_hbm, raw_ref):
        def body(i_vmem, o_vmem):
            pltpu.sync_copy(tab_hbm.at[i_vmem.at[0, 0, 0]], raw_ref)
            for d in range(D):
                for g in range(128 // _LANES):
                    rows = jnp.arange(g * _LANES, (g + 1) * _LANES,
                                      dtype=jnp.int32)
                    cols = jnp.full((_LANES,), d, dtype=jnp.int32)
                    o_vmem.at[0, d // 8, 0, d % 8, pl.ds(g * _LANES, _LANES)][...] = (
                        plsc.load_gather(raw_ref, [rows, cols])
                    )

        pltpu.emit_pipeline(
            body,
            grid=(LB, BB, 8),
            in_specs=[pl.BlockSpec((1, 1, 1, 128),
                                   index_map=lambda i, j, k: (i, j, k, 0))],
            out_specs=[pl.BlockSpec((1, D // 8, 1, 8, 128),
                                    index_map=lambda i, j, k: (i * 8 + k, 0, j, 0, 0))],
            core_axis_name=("c", "s"),
            dimension_semantics=(pltpu.PARALLEL, pltpu.PARALLEL, pltpu.PARALLEL),
        )(idx_hbm, out_hbm)

    out5 = gather_kernel(embedding_table, idx4)
    return out5.transpose(2, 4, 0, 1, 3).reshape(B, L, D)


# speedup vs baseline: 1.1879x; 1.1879x over previous
import functools

import jax
import jax.numpy as jnp
from jax.experimental import pallas as pl
from jax.experimental.pallas import tpu as pltpu
from jax.experimental.pallas import tpu_sc as plsc

_LANES = 16
_Q = 4


def kernel(token_ids, embedding_table):
    B, L = token_ids.shape
    D = embedding_table.shape[1]
    LB, BB = L // 8, B // 128

    idx4 = token_ids.reshape(BB, 128, LB, 8).transpose(2, 0, 3, 1)

    mesh = plsc.VectorSubcoreMesh(core_axis_name="c", subcore_axis_name="s")

    @functools.partial(
        pl.kernel,
        mesh=mesh,
        out_type=jax.ShapeDtypeStruct((L, D // 8, BB, 8, 128), jnp.float32),
        scratch_types=[pltpu.VMEM((_Q * 128, D), jnp.float32),
                       pltpu.SemaphoreType.DMA],
        compiler_params=pltpu.CompilerParams(use_tc_tiling_on_sc=False,
                                             needs_layout_passes=False),
    )
    def gather_kernel(tab_hbm, idx_hbm, out_hbm, raw_ref, sem):
        rows = [jnp.arange(g * _LANES, (g + 1) * _LANES, dtype=jnp.int32)
                for g in range(128 // _LANES)]
        cols = [jnp.full((_LANES,), d, dtype=jnp.int32) for d in range(D)]

        def body(i_vmem, o_vmem):
            copies = [
                pltpu.async_copy(
                    tab_hbm.at[i_vmem.at[0, 0, q]],
                    raw_ref.at[pl.ds(q * 128, 128)],
                    sem,
                )
                for q in range(_Q)
            ]
            for c in copies:
                c.wait()
            for q in range(_Q):
                for d in range(D):
                    vals = [
                        plsc.load_gather(raw_ref,
                                         [rows[g] + q * 128, cols[d]])
                        for g in range(128 // _LANES)
                    ]
                    for g in range(128 // _LANES):
                        o_vmem.at[q, d // 8, 0, d % 8,
                                  pl.ds(g * _LANES, _LANES)][...] = vals[g]

        pltpu.emit_pipeline(
            body,
            grid=(LB, BB, 8 // _Q),
            in_specs=[pl.BlockSpec((1, 1, _Q, 128),
                                   index_map=lambda i, j, k: (i, j, k, 0))],
            out_specs=[pl.BlockSpec((_Q, D // 8, 1, 8, 128),
                                    index_map=lambda i, j, k: (i * (8 // _Q) + k, 0, j, 0, 0))],
            core_axis_name=("c", "s"),
            dimension_semantics=(pltpu.PARALLEL, pltpu.PARALLEL, pltpu.PARALLEL),
        )(idx_hbm, out_hbm)

    out5 = gather_kernel(embedding_table, idx4)
    return out5.transpose(2, 4, 0, 1, 3).reshape(B, L, D)


# speedup vs baseline: 1.4402x; 1.2124x over previous
import functools

import jax
import jax.numpy as jnp
from jax.experimental import pallas as pl
from jax.experimental.pallas import tpu as pltpu
from jax.experimental.pallas import tpu_sc as plsc

_LANES = 16
_Q = 4


def _compact_table(table_t):
    Dd, V = table_t.shape
    C = 16384

    def body(x_ref, o_ref):
        o_ref[:, :Dd] = x_ref[...].T
        o_ref[:, Dd:] = jnp.zeros((C, 128 - Dd), jnp.float32)

    return pl.pallas_call(
        body,
        grid=((V + C - 1) // C,),
        in_specs=[pl.BlockSpec((Dd, C), lambda i: (0, i))],
        out_specs=pl.BlockSpec((C, 128), lambda i: (i, 0)),
        out_shape=jax.ShapeDtypeStruct((V, 128), jnp.float32),
        compiler_params=pltpu.CompilerParams(
            dimension_semantics=("parallel",)),
    )(table_t)


def kernel(token_ids, embedding_table):
    B, L = token_ids.shape
    D = embedding_table.shape[1]
    LB, BB = L // 8, B // 128

    idx4 = token_ids.reshape(BB, 128, LB, 8).transpose(2, 0, 3, 1)

    tab_lin = _compact_table(embedding_table.T)

    mesh = plsc.VectorSubcoreMesh(core_axis_name="c", subcore_axis_name="s")

    @functools.partial(
        pl.kernel,
        mesh=mesh,
        out_type=jax.ShapeDtypeStruct((L, D // 8, BB, 8, 128), jnp.float32),
        scratch_types=[pltpu.VMEM((_Q * 128, 128), jnp.float32),
                       pltpu.SemaphoreType.DMA],
        compiler_params=pltpu.CompilerParams(use_tc_tiling_on_sc=False,
                                             needs_layout_passes=False),
    )
    def gather_kernel(tab_hbm, idx_hbm, out_hbm, raw_ref, sem):
        rows = [jnp.arange(g * _LANES, (g + 1) * _LANES, dtype=jnp.int32)
                for g in range(128 // _LANES)]
        cols = [jnp.full((_LANES,), d, dtype=jnp.int32) for d in range(D)]

        def body(i_vmem, o_vmem):
            copies = [
                pltpu.async_copy(
                    tab_hbm.at[i_vmem.at[0, 0, q]],
                    raw_ref.at[pl.ds(q * 128, 128)],
                    sem,
                )
                for q in range(_Q)
            ]
            for c in copies:
                c.wait()
            for q in range(_Q):
                for d in range(D):
                    vals = [
                        plsc.load_gather(raw_ref,
                                         [rows[g] + q * 128, cols[d]])
                        for g in range(128 // _LANES)
                    ]
                    for g in range(128 // _LANES):
                        o_vmem.at[q, d // 8, 0, d % 8,
                                  pl.ds(g * _LANES, _LANES)][...] = vals[g]

        pltpu.emit_pipeline(
            body,
            grid=(LB, BB, 8 // _Q),
            in_specs=[pl.BlockSpec((1, 1, _Q, 128),
                                   index_map=lambda i, j, k: (i, j, k, 0))],
            out_specs=[pl.BlockSpec((_Q, D // 8, 1, 8, 128),
                                    index_map=lambda i, j, k: (i * (8 // _Q) + k, 0, j, 0, 0))],
            core_axis_name=("c", "s"),
            dimension_semantics=(pltpu.PARALLEL, pltpu.PARALLEL, pltpu.PARALLEL),
        )(idx_hbm, out_hbm)

    out5 = gather_kernel(tab_lin, idx4)
    return out5.transpose(2, 4, 0, 1, 3).reshape(B, L, D)


# speedup vs baseline: 1.6182x; 1.1236x over previous
import functools

import jax
import jax.numpy as jnp
from jax.experimental import pallas as pl
from jax.experimental.pallas import tpu as pltpu
from jax.experimental.pallas import tpu_sc as plsc

_LANES = 16
_Q = 4


def _compact_table(table_t):
    Dd, V = table_t.shape
    C = 16384

    def body(x_ref, o_ref):
        o_ref[:, :Dd] = x_ref[...].T
        o_ref[:, Dd:] = jnp.zeros((C, 128 - Dd), jnp.float32)

    return pl.pallas_call(
        body,
        grid=((V + C - 1) // C,),
        in_specs=[pl.BlockSpec((Dd, C), lambda i: (0, i))],
        out_specs=pl.BlockSpec((C, 128), lambda i: (i, 0)),
        out_shape=jax.ShapeDtypeStruct((V, 128), jnp.float32),
        compiler_params=pltpu.CompilerParams(
            dimension_semantics=("parallel",)),
    )(table_t)


def kernel(token_ids, embedding_table):
    B, L = token_ids.shape
    D = embedding_table.shape[1]
    LB, BB = L // 8, B // 128

    idx4 = token_ids.reshape(BB, 128, LB, 8).transpose(2, 0, 3, 1)

    tab_lin = _compact_table(embedding_table.T)

    mesh = plsc.VectorSubcoreMesh(core_axis_name="c", subcore_axis_name="s")

    @functools.partial(
        pl.kernel,
        mesh=mesh,
        out_type=jax.ShapeDtypeStruct((L, D // 8, BB, 8, 128), jnp.float32),
        scratch_types=[pltpu.VMEM((_Q * 128, 128), jnp.float32),
                       pltpu.SemaphoreType.DMA],
        compiler_params=pltpu.CompilerParams(use_tc_tiling_on_sc=False,
                                             needs_layout_passes=False),
    )
    def gather_kernel(tab_hbm, idx_hbm, out_hbm, raw_ref, sem):
        rows = [jnp.arange(g * _LANES, (g + 1) * _LANES, dtype=jnp.int32)
                for g in range(128 // _LANES)]
        cols = [jnp.full((_LANES,), d, dtype=jnp.int32) for d in range(D)]

        def body(i_vmem, o_vmem):
            copies = [
                pltpu.async_copy(
                    tab_hbm.at[i_vmem.at[0, 0, q]],
                    raw_ref.at[pl.ds(q * 128, 128)],
                    sem,
                )
                for q in range(_Q)
            ]
            for q in range(_Q):
                copies[q].wait()
                for d in range(D):
                    vals = [
                        plsc.load_gather(raw_ref,
                                         [rows[g] + q * 128, cols[d]])
                        for g in range(128 // _LANES)
                    ]
                    for g in range(128 // _LANES):
                        o_vmem.at[q, d // 8, 0, d % 8,
                                  pl.ds(g * _LANES, _LANES)][...] = vals[g]

        pltpu.emit_pipeline(
            body,
            grid=(LB, BB, 8 // _Q),
            in_specs=[pl.BlockSpec((1, 1, _Q, 128),
                                   index_map=lambda i, j, k: (i, j, k, 0))],
            out_specs=[pl.BlockSpec((_Q, D // 8, 1, 8, 128),
                                    index_map=lambda i, j, k: (i * (8 // _Q) + k, 0, j, 0, 0))],
            core_axis_name=("c", "s"),
            dimension_semantics=(pltpu.PARALLEL, pltpu.PARALLEL, pltpu.PARALLEL),
        )(idx_hbm, out_hbm)

    out5 = gather_kernel(tab_lin, idx4)
    return out5.transpose(2, 4, 0, 1, 3).reshape(B, L, D)
